# Initial kernel scaffold; baseline (speedup 1.0000x reference)
import jax
import jax.numpy as jnp
from jax.experimental import pallas as pl


def kernel(x, edge_index, edge_attr, batch, We, be, W, b, Wf1, bf1, Wf2, bf2):
    raise NotImplementedError("write your pallas kernel here")


# speedup vs baseline: 2.2974x; 2.2974x over previous
import functools

import jax
import jax.numpy as jnp
from jax import lax
from jax.experimental import pallas as pl
from jax.experimental.pallas import tpu as pltpu
from jax.experimental.pallas import tpu_sc as plsc

N = 10000
E = 320000
D = 128
DE = 16
L = 3
G = 128
OUT = 16

NC = 2
NS = 16
NW = NC * NS
CH = 128
NCH = 79
EPW = CH * NCH
E_PAD = EPW * NW
N_ACC = 10240
RPS = N_ACC // NS
EB = 4096
BR = 1000
NB = N // BR


def _enc_body(ea_ref, we_ref, be_ref, out_ref):
    out_ref[0] = (
        jnp.dot(ea_ref[...], we_ref[0], preferred_element_type=jnp.float32)
        + be_ref[0]
    )


_enc = pl.pallas_call(
    _enc_body,
    grid=(L, E_PAD // EB),
    in_specs=[
        pl.BlockSpec((EB, DE), lambda l, i: (i, 0)),
        pl.BlockSpec((1, DE, D), lambda l, i: (l, 0, 0)),
        pl.BlockSpec((1, 1, D), lambda l, i: (l, 0, 0)),
    ],
    out_specs=pl.BlockSpec((1, EB, D), lambda l, i: (l, i, 0)),
    out_shape=jax.ShapeDtypeStruct((L, E_PAD, D), jnp.float32),
)


def _sc_body(h_hbm, e_hbm, src_hbm, dst_hbm, out_hbm,
             src_v, dst_v, e_v, sem, acc_sh, *, l):
    cid = lax.axis_index("c")
    sid = lax.axis_index("s")
    w = cid * NS + sid
    r0 = sid * RPS

    def _zrow(r, c):
        for k in range(D // 16):
            e_v[r, pl.ds(k * 16, 16)] = jnp.zeros((16,), jnp.float32)
        return c
    lax.fori_loop(0, CH, _zrow, 0)

    def _zacc(i, c):
        pltpu.sync_copy(e_v, acc_sh.at[pl.ds(r0 + i * CH, CH)])
        return c
    lax.fori_loop(0, RPS // CH, _zacc, 0)
    plsc.subcore_barrier()

    def _chunk(i, c):
        base = w * EPW + i * CH
        pltpu.sync_copy(src_hbm.at[pl.ds(base, CH)], src_v)
        pltpu.sync_copy(dst_hbm.at[pl.ds(base, CH)], dst_v)
        pltpu.sync_copy(e_hbm.at[l, pl.ds(base, CH)], e_v)
        pltpu.async_copy(h_hbm.at[src_v], e_v, sem, add=True).wait()

        def _relu(r, cc):
            for k in range(D // 16):
                v = e_v[r, pl.ds(k * 16, 16)]
                e_v[r, pl.ds(k * 16, 16)] = jnp.maximum(v, 0.0)
            return cc
        lax.fori_loop(0, CH, _relu, 0)

        pltpu.sync_copy(e_v, acc_sh.at[dst_v], add=True)
        return c
    lax.fori_loop(0, NCH, _chunk, 0)
    plsc.subcore_barrier()

    def _wout(i, c):
        r = r0 + i * CH
        pltpu.sync_copy(acc_sh.at[pl.ds(r, CH)], e_v)

        @pl.when(cid == 0)
        def _():
            pltpu.sync_copy(e_v, out_hbm.at[0, pl.ds(r, CH)])

        @pl.when(cid == 1)
        def _():
            pltpu.sync_copy(e_v, out_hbm.at[1, pl.ds(r, CH)])
        return c
    lax.fori_loop(0, RPS // CH, _wout, 0)


def _make_sc(l):
    return pl.kernel(
        functools.partial(_sc_body, l=l),
        out_type=jax.ShapeDtypeStruct((NC, N_ACC, D), jnp.float32),
        mesh=plsc.VectorSubcoreMesh(core_axis_name="c", subcore_axis_name="s"),
        scratch_types=[
            pltpu.VMEM((CH,), jnp.int32),
            pltpu.VMEM((CH,), jnp.int32),
            pltpu.VMEM((CH, D), jnp.float32),
            pltpu.SemaphoreType.DMA,
            pltpu.VMEM_SHARED((N_ACC, D), jnp.float32),
        ],
    )


_sc_layers = [_make_sc(l) for l in range(L)]


def _upd_body(h_ref, a_ref, w_ref, b_ref, o_ref):
    s = h_ref[...] + a_ref[0] + a_ref[1]
    o_ref[...] = jnp.maximum(
        jnp.dot(s, w_ref[0], preferred_element_type=jnp.float32) + b_ref[0],
        0.0,
    )


def _make_upd(l):
    return pl.pallas_call(
        _upd_body,
        grid=(NB,),
        in_specs=[
            pl.BlockSpec((BR, D), lambda i: (i, 0)),
            pl.BlockSpec((2, BR, D), lambda i: (0, i, 0)),
            pl.BlockSpec((1, D, D), lambda i, l=l: (l, 0, 0)),
            pl.BlockSpec((1, 1, D), lambda i, l=l: (l, 0, 0)),
        ],
        out_specs=pl.BlockSpec((BR, D), lambda i: (i, 0)),
        out_shape=jax.ShapeDtypeStruct((N, D), jnp.float32),
    )


_upd_layers = [_make_upd(l) for l in range(L)]


def _pool_body(h_ref, b_ref, wf1_ref, bf1_ref, wf2_ref, bf2_ref, y_ref, acc):
    i = pl.program_id(0)

    @pl.when(i == 0)
    def _():
        acc[...] = jnp.zeros((G, D), jnp.float32)

    ids = b_ref[0]
    gi = lax.broadcasted_iota(jnp.int32, (G, BR), 0)
    sel = jnp.where(ids == gi, 1.0, 0.0).astype(jnp.float32)
    acc[...] += jnp.dot(sel, h_ref[...], preferred_element_type=jnp.float32)

    @pl.when(i == NB - 1)
    def _():
        g = jnp.maximum(acc[...], 0.0)
        y1 = jnp.dot(g, wf1_ref[...], preferred_element_type=jnp.float32) + bf1_ref[...]
        y_ref[...] = jnp.dot(y1, wf2_ref[...], preferred_element_type=jnp.float32) + bf2_ref[...]


_pool = pl.pallas_call(
    _pool_body,
    grid=(NB,),
    in_specs=[
        pl.BlockSpec((BR, D), lambda i: (i, 0)),
        pl.BlockSpec((1, 1, BR), lambda i: (i, 0, 0)),
        pl.BlockSpec((D, D), lambda i: (0, 0)),
        pl.BlockSpec((1, D), lambda i: (0, 0)),
        pl.BlockSpec((D, OUT), lambda i: (0, 0)),
        pl.BlockSpec((1, OUT), lambda i: (0, 0)),
    ],
    out_specs=pl.BlockSpec((G, OUT), lambda i: (0, 0)),
    out_shape=jax.ShapeDtypeStruct((G, OUT), jnp.float32),
    scratch_shapes=[pltpu.VMEM((G, D), jnp.float32)],
)


def kernel(x, edge_index, edge_attr, batch, We, be, W, b, Wf1, bf1, Wf2, bf2):
    pad = E_PAD - E
    src_p = jnp.pad(edge_index[0], (0, pad))
    dst_p = jnp.pad(edge_index[1], (0, pad), constant_values=N)
    ea_p = jnp.pad(edge_attr, ((0, pad), (0, 0)))

    e_all = _enc(ea_p, We, be.reshape(L, 1, D))

    h = x
    for l in range(L):
        agg = _sc_layers[l](h, e_all, src_p, dst_p)
        h = _upd_layers[l](h, agg, W, b.reshape(L, 1, D))

    batch3 = batch.reshape(NB, 1, BR)
    return _pool(h, batch3, Wf1.reshape(D, D), bf1.reshape(1, D),
                 Wf2.reshape(D, OUT), bf2.reshape(1, OUT))
